# baseline (speedup 1.0000x reference)
import functools

import jax
import jax.numpy as jnp
from jax import lax
from jax.experimental import pallas as pl
from jax.experimental.pallas import tpu as pltpu
from jax.experimental.pallas import tpu_sc as plsc

NC = 2
NS = 16
L = 16
NW = NC * NS
ROWS = 128
COLS = 32768
SC_ROWS = NW
TC_ROWS = ROWS - SC_ROWS
U = 8
STEPS = COLS // (U * L)
I32_MAX = 2**31 - 1

_DNUMS = lax.GatherDimensionNumbers(
    offset_dims=(), collapsed_slice_dims=(0,), start_index_map=(0,)
)


def _shuf(v, perm):
    return lax.gather(
        v, perm[:, None], _DNUMS, slice_sizes=(1,),
        mode=lax.GatherScatterMode.PROMISE_IN_BOUNDS,
    )


def _row_argmax(row_ref, lane_iota):
    ninf = jnp.full((L,), -jnp.inf, jnp.float32)
    zero = jnp.zeros((L,), jnp.int32)

    def body(i, carry):
        maxs, steps = carry
        ib = jnp.broadcast_to(i, (L,)).astype(jnp.int32)
        new_maxs = []
        new_steps = []
        base = i * (U * L)
        for k in range(U):
            v = row_ref[pl.ds(base + k * L, L)]
            take = v > maxs[k]
            new_maxs.append(jnp.where(take, v, maxs[k]))
            new_steps.append(jnp.where(take, ib, steps[k]))
        return tuple(new_maxs), tuple(new_steps)

    maxs, steps = lax.fori_loop(
        0, STEPS, body, ((ninf,) * U, (zero,) * U), unroll=False
    )

    m = maxs[0]
    idx = steps[0] * (U * L) + lane_iota
    for k in range(1, U):
        idx_k = steps[k] * (U * L) + (k * L) + lane_iota
        take = (maxs[k] > m) | ((maxs[k] == m) & (idx_k < idx))
        m = jnp.where(take, maxs[k], m)
        idx = jnp.where(take, idx_k, idx)

    for sh in (1, 2, 4, 8):
        perm = lane_iota ^ sh
        m2 = _shuf(m, perm)
        idx2 = _shuf(idx, perm)
        take = (m2 > m) | ((m2 == m) & (idx2 < idx))
        m = jnp.where(take, m2, m)
        idx = jnp.where(take, idx2, idx)
    return idx


_mesh = plsc.VectorSubcoreMesh(core_axis_name="c", subcore_axis_name="s")


@functools.partial(
    pl.kernel,
    out_type=jax.ShapeDtypeStruct((NW, L), jnp.int32),
    mesh=_mesh,
    scratch_types=[
        pltpu.VMEM((COLS,), jnp.float32),
        pltpu.VMEM((L,), jnp.int32),
        pltpu.SemaphoreType.DMA,
    ],
)
def _argmax_sc(x_hbm, out_hbm, buf, res_v, sem):
    wid = lax.axis_index("s") * NC + lax.axis_index("c")
    lane_iota = lax.iota(jnp.int32, L)
    cp = pltpu.make_async_copy(x_hbm.at[wid], buf, sem)
    cp.start()
    cp.wait()
    res_v[...] = _row_argmax(buf, lane_iota)
    pltpu.sync_copy(res_v, out_hbm.at[wid])


BS = 8192
SUB = BS // 128
GRID = COLS // BS
RB = 32
NRB = TC_ROWS // RB


def _tc_body(x_ref, out_ref, amax_ref, astep_ref):
    j = pl.program_id(1)

    @pl.when(j == 0)
    def _init():
        amax_ref[...] = jnp.full((RB, 128), -jnp.inf, jnp.float32)
        astep_ref[...] = jnp.zeros((RB, 128), jnp.int32)

    amax = amax_ref[...]
    astep = astep_ref[...]
    for s in range(SUB):
        v = x_ref[:, s * 128:(s + 1) * 128]
        step = j * SUB + s
        take = v > amax
        amax = jnp.where(take, v, amax)
        astep = jnp.where(take, step, astep)
    amax_ref[...] = amax
    astep_ref[...] = astep

    @pl.when(j == GRID - 1)
    def _finish():
        lanes = lax.broadcasted_iota(jnp.int32, (RB, 128), 1)
        idx = astep * 128 + lanes
        gmax = jnp.max(amax, axis=1, keepdims=True)
        cand = jnp.where(amax == gmax, idx, I32_MAX)
        out_ref[...] = jnp.min(cand, axis=1).reshape(1, 1, RB)


_argmax_tc = pl.pallas_call(
    _tc_body,
    grid=(NRB, GRID),
    in_specs=[pl.BlockSpec((RB, BS), lambda i, j: (i + SC_ROWS // RB, j))],
    out_specs=pl.BlockSpec((1, 1, RB), lambda i, j: (i, 0, 0)),
    out_shape=jax.ShapeDtypeStruct((NRB, 1, RB), jnp.int32),
    scratch_shapes=[
        pltpu.VMEM((RB, 128), jnp.float32),
        pltpu.VMEM((RB, 128), jnp.int32),
    ],
)


@jax.jit
def kernel(x):
    sc_out = _argmax_sc(x)
    tc_out = _argmax_tc(x)
    return jnp.concatenate([sc_out[:, 0], tc_out.reshape(TC_ROWS)])


# speedup vs baseline: 3.4354x; 3.4354x over previous
import jax
import jax.numpy as jnp
from jax import lax
from jax.experimental import pallas as pl
from jax.experimental.pallas import tpu as pltpu

ROWS = 128
COLS = 32768
HALF = COLS // 2
BSH = 4096
SUBH = BSH // 128
GRID = HALF // BSH
HALFSTEP = HALF // 128
I32_MAX = 2**31 - 1


def _tc_body(a_ref, b_ref, out_ref, amax_a, astep_a, amax_b, astep_b):
    j = pl.program_id(0)

    @pl.when(j == 0)
    def _init():
        amax_a[...] = jnp.full((ROWS, 128), -jnp.inf, jnp.float32)
        astep_a[...] = jnp.zeros((ROWS, 128), jnp.int32)
        amax_b[...] = jnp.full((ROWS, 128), -jnp.inf, jnp.float32)
        astep_b[...] = jnp.zeros((ROWS, 128), jnp.int32)

    for ref, mref, sref in ((a_ref, amax_a, astep_a), (b_ref, amax_b, astep_b)):
        amax = mref[...]
        astep = sref[...]
        for s in range(SUBH):
            v = ref[:, s * 128:(s + 1) * 128]
            step = j * SUBH + s
            take = v > amax
            amax = jnp.where(take, v, amax)
            astep = jnp.where(take, step, astep)
        mref[...] = amax
        sref[...] = astep

    @pl.when(j == GRID - 1)
    def _finish():
        lanes = lax.broadcasted_iota(jnp.int32, (ROWS, 128), 1)
        idx_a = astep_a[...] * 128 + lanes
        idx_b = (astep_b[...] + HALFSTEP) * 128 + lanes
        ma, mb = amax_a[...], amax_b[...]
        take_b = mb > ma
        m = jnp.where(take_b, mb, ma)
        idx = jnp.where(take_b, idx_b, idx_a)
        gmax = jnp.max(m, axis=1, keepdims=True)
        cand = jnp.where(m == gmax, idx, I32_MAX)
        out_ref[...] = jnp.min(cand, axis=1)


_argmax_tc = pl.pallas_call(
    _tc_body,
    grid=(GRID,),
    in_specs=[
        pl.BlockSpec((ROWS, BSH), lambda j: (0, j)),
        pl.BlockSpec((ROWS, BSH), lambda j: (0, j + GRID)),
    ],
    out_specs=pl.BlockSpec((ROWS,), lambda j: (0,)),
    out_shape=jax.ShapeDtypeStruct((ROWS,), jnp.int32),
    scratch_shapes=[
        pltpu.VMEM((ROWS, 128), jnp.float32),
        pltpu.VMEM((ROWS, 128), jnp.int32),
        pltpu.VMEM((ROWS, 128), jnp.float32),
        pltpu.VMEM((ROWS, 128), jnp.int32),
    ],
)


@jax.jit
def kernel(x):
    return _argmax_tc(x, x)
